# baseline (speedup 1.0000x reference)
import jax
import jax.numpy as jnp
from jax import lax
from jax.experimental import pallas as pl
from jax.experimental.pallas import tpu as pltpu
from jax.experimental.pallas import tpu_sc as plsc

F, B, L, V, D = 26, 1024, 20, 100000, 128
N = F * B * L
NC, NS = 2, 16
NW = NC * NS
LANES = 16

CHUNK = 80
SLICES = ((0, 80),)
NCHUNK = (N // NW) // CHUNK
BAGS = CHUNK // L
UNITS_PER_F = B // BAGS
ROWS_PER_W = N // NW
NT = 4
AHEAD = 2


def _sc_body(idx_hbm, ebc1_hbm, ec1_hbm, ebc2_hbm, ec2_hbm,
             o_ebc1, o_ec1, o_ebc2, o_ec2,
             rows0, rows1, rows2, rows3, idx_all, acc0, acc1,
             gsem0, gsem1, gsem2, gsem3, wsem0, wsem1, wsem2, wsem3):
    wid = lax.axis_index("s") * NC + lax.axis_index("c")
    rows = (rows0, rows1, rows2, rows3)
    accb = {1: acc0, 3: acc1}
    gsem = (gsem0, gsem1, gsem2, gsem3)
    wsem = (wsem0, wsem1, wsem2, wsem3)

    pltpu.sync_copy(idx_hbm.at[pl.ds(wid * ROWS_PER_W, ROWS_PER_W)], idx_all)

    tables = (ec1_hbm, ebc1_hbm, ec2_hbm, ebc2_hbm)
    outs = (o_ec1, o_ebc1, o_ec2, o_ebc2)

    def fire(b, i, table_hbm):
        for (off, sz) in SLICES:
            o = pl.multiple_of(i * CHUNK + off, 8)
            pltpu.async_copy(table_hbm.at[idx_all.at[pl.ds(o, sz)]],
                             rows[b].at[pl.ds(off, sz)], gsem[b])

    def drain_gather(b, table_hbm):
        pltpu.make_async_copy(table_hbm.at[pl.ds(0, CHUNK)], rows[b],
                              gsem[b]).wait()

    def process(t, i):
        out_hbm = outs[t]
        if t % 2 == 0:
            o0 = pl.multiple_of(wid * ROWS_PER_W + i * CHUNK, CHUNK)
            pltpu.async_copy(rows[t], out_hbm.at[pl.ds(o0, CHUNK)], wsem[t])
        else:
            acc = accb[t]

            def pool(bag, c2):
                base = bag * L
                for c in range(D // LANES):
                    a = rows[t][base, pl.ds(c * LANES, LANES)]
                    for l in range(1, L):
                        a = a + rows[t][base + l, pl.ds(c * LANES, LANES)]
                    acc[bag, pl.ds(c * LANES, LANES)] = a
                return c2

            lax.fori_loop(0, BAGS, pool, 0)
            u = wid * NCHUNK + i
            f = u // UNITS_PER_F
            bc = u % UNITS_PER_F
            b0 = pl.multiple_of(bc * BAGS, BAGS)
            col0 = pl.multiple_of(f * D, D)
            pltpu.async_copy(acc, out_hbm.at[pl.ds(b0, BAGS), pl.ds(col0, D)],
                             wsem[t])

    def drain_write(t):
        out_hbm = outs[t]
        if t % 2 == 0:
            pltpu.make_async_copy(rows[t], out_hbm.at[pl.ds(0, CHUNK)],
                                  wsem[t]).wait()
        else:
            pltpu.make_async_copy(accb[t],
                                  out_hbm.at[pl.ds(0, BAGS), pl.ds(0, D)],
                                  wsem[t]).wait()

    for s in range(AHEAD):
        fire(s % NT, 0, tables[s % NT])

    def outer(i, carry):
        for t in range(NT):
            ft = (t + AHEAD) % NT
            fi = i + 1 if t + AHEAD >= NT else i

            if t + AHEAD < NT:
                @pl.when(i >= 1)
                def _():
                    drain_write(ft)
                fire(ft, fi, tables[ft])
            else:
                drain_write(ft)

                @pl.when(fi < NCHUNK)
                def _():
                    fire(ft, fi, tables[ft])

            drain_gather(t, tables[t])
            process(t, i)
        return carry

    lax.fori_loop(0, NCHUNK, outer, 0)
    for t in range(NT - AHEAD, NT):
        drain_write(t)


@jax.jit
def kernel(indices, ebc1_table, ec1_table, ebc2_table, ec2_table):
    idx1d = indices.reshape(N)
    mesh = plsc.VectorSubcoreMesh(core_axis_name="c", subcore_axis_name="s",
                                  num_cores=NC, num_subcores=NS)
    out_type = (
        jax.ShapeDtypeStruct((B, F * D), jnp.float32),
        jax.ShapeDtypeStruct((N, D), jnp.float32),
        jax.ShapeDtypeStruct((B, F * D), jnp.float32),
        jax.ShapeDtypeStruct((N, D), jnp.float32),
    )
    scratch = [
        pltpu.VMEM((CHUNK, D), jnp.float32),
        pltpu.VMEM((CHUNK, D), jnp.float32),
        pltpu.VMEM((CHUNK, D), jnp.float32),
        pltpu.VMEM((CHUNK, D), jnp.float32),
        pltpu.VMEM((ROWS_PER_W,), jnp.int32),
        pltpu.VMEM((BAGS, D), jnp.float32),
        pltpu.VMEM((BAGS, D), jnp.float32),
        pltpu.SemaphoreType.DMA,
        pltpu.SemaphoreType.DMA,
        pltpu.SemaphoreType.DMA,
        pltpu.SemaphoreType.DMA,
        pltpu.SemaphoreType.DMA,
        pltpu.SemaphoreType.DMA,
        pltpu.SemaphoreType.DMA,
        pltpu.SemaphoreType.DMA,
    ]
    fn = pl.kernel(_sc_body, out_type=out_type, mesh=mesh,
                   scratch_types=scratch)
    return fn(idx1d, ebc1_table, ec1_table, ebc2_table, ec2_table)


# speedup vs baseline: 1.0626x; 1.0626x over previous
import jax
import jax.numpy as jnp
from jax import lax
from jax.experimental import pallas as pl
from jax.experimental.pallas import tpu as pltpu
from jax.experimental.pallas import tpu_sc as plsc

F, B, L, V, D = 26, 1024, 20, 100000, 128
N = F * B * L
NC, NS = 2, 16
NW = NC * NS
LANES = 16

CHUNK = 160
SLICES = ((0, 80), (80, 80))
NCHUNK = (N // NW) // CHUNK
BAGS = CHUNK // L
UNITS_PER_F = B // BAGS
ROWS_PER_W = N // NW
NT = 4
AHEAD = 2


def _sc_body(idx_hbm, ebc1_hbm, ec1_hbm, ebc2_hbm, ec2_hbm,
             o_ebc1, o_ec1, o_ebc2, o_ec2,
             rows0, rows1, rows2, rows3, idx_all, acc0, acc1,
             gsem0, gsem1, gsem2, gsem3, wsem0, wsem1, wsem2, wsem3):
    wid = lax.axis_index("s") * NC + lax.axis_index("c")
    rows = (rows0, rows1, rows2, rows3)
    accb = {1: acc0, 3: acc1}
    gsem = (gsem0, gsem1, gsem2, gsem3)
    wsem = (wsem0, wsem1, wsem2, wsem3)

    pltpu.sync_copy(idx_hbm.at[pl.ds(wid * ROWS_PER_W, ROWS_PER_W)], idx_all)

    tables = (ec1_hbm, ebc1_hbm, ec2_hbm, ebc2_hbm)
    outs = (o_ec1, o_ebc1, o_ec2, o_ebc2)

    def fire(b, i, table_hbm):
        for (off, sz) in SLICES:
            o = pl.multiple_of(i * CHUNK + off, 8)
            pltpu.async_copy(table_hbm.at[idx_all.at[pl.ds(o, sz)]],
                             rows[b].at[pl.ds(off, sz)], gsem[b])

    def drain_gather(b, table_hbm):
        pltpu.make_async_copy(table_hbm.at[pl.ds(0, CHUNK)], rows[b],
                              gsem[b]).wait()

    def process(t, i):
        out_hbm = outs[t]
        if t % 2 == 0:
            o0 = pl.multiple_of(wid * ROWS_PER_W + i * CHUNK, CHUNK)
            pltpu.async_copy(rows[t], out_hbm.at[pl.ds(o0, CHUNK)], wsem[t])
        else:
            acc = accb[t]

            def pool(bag, c2):
                base = bag * L
                for c in range(D // LANES):
                    a = rows[t][base, pl.ds(c * LANES, LANES)]
                    for l in range(1, L):
                        a = a + rows[t][base + l, pl.ds(c * LANES, LANES)]
                    acc[bag, pl.ds(c * LANES, LANES)] = a
                return c2

            lax.fori_loop(0, BAGS, pool, 0)
            u = wid * NCHUNK + i
            f = u // UNITS_PER_F
            bc = u % UNITS_PER_F
            b0 = pl.multiple_of(bc * BAGS, BAGS)
            col0 = pl.multiple_of(f * D, D)
            pltpu.async_copy(acc, out_hbm.at[pl.ds(b0, BAGS), pl.ds(col0, D)],
                             wsem[t])

    def drain_write(t):
        out_hbm = outs[t]
        if t % 2 == 0:
            pltpu.make_async_copy(rows[t], out_hbm.at[pl.ds(0, CHUNK)],
                                  wsem[t]).wait()
        else:
            pltpu.make_async_copy(accb[t],
                                  out_hbm.at[pl.ds(0, BAGS), pl.ds(0, D)],
                                  wsem[t]).wait()

    for s in range(AHEAD):
        fire(s % NT, 0, tables[s % NT])

    def outer(i, carry):
        for t in range(NT):
            ft = (t + AHEAD) % NT
            fi = i + 1 if t + AHEAD >= NT else i

            if t + AHEAD < NT:
                @pl.when(i >= 1)
                def _():
                    drain_write(ft)
                fire(ft, fi, tables[ft])
            else:
                drain_write(ft)

                @pl.when(fi < NCHUNK)
                def _():
                    fire(ft, fi, tables[ft])

            drain_gather(t, tables[t])
            process(t, i)
        return carry

    lax.fori_loop(0, NCHUNK, outer, 0)
    for t in range(NT - AHEAD, NT):
        drain_write(t)


@jax.jit
def kernel(indices, ebc1_table, ec1_table, ebc2_table, ec2_table):
    idx1d = indices.reshape(N)
    mesh = plsc.VectorSubcoreMesh(core_axis_name="c", subcore_axis_name="s",
                                  num_cores=NC, num_subcores=NS)
    out_type = (
        jax.ShapeDtypeStruct((B, F * D), jnp.float32),
        jax.ShapeDtypeStruct((N, D), jnp.float32),
        jax.ShapeDtypeStruct((B, F * D), jnp.float32),
        jax.ShapeDtypeStruct((N, D), jnp.float32),
    )
    scratch = [
        pltpu.VMEM((CHUNK, D), jnp.float32),
        pltpu.VMEM((CHUNK, D), jnp.float32),
        pltpu.VMEM((CHUNK, D), jnp.float32),
        pltpu.VMEM((CHUNK, D), jnp.float32),
        pltpu.VMEM((ROWS_PER_W,), jnp.int32),
        pltpu.VMEM((BAGS, D), jnp.float32),
        pltpu.VMEM((BAGS, D), jnp.float32),
        pltpu.SemaphoreType.DMA,
        pltpu.SemaphoreType.DMA,
        pltpu.SemaphoreType.DMA,
        pltpu.SemaphoreType.DMA,
        pltpu.SemaphoreType.DMA,
        pltpu.SemaphoreType.DMA,
        pltpu.SemaphoreType.DMA,
        pltpu.SemaphoreType.DMA,
    ]
    fn = pl.kernel(_sc_body, out_type=out_type, mesh=mesh,
                   scratch_types=scratch)
    return fn(idx1d, ebc1_table, ec1_table, ebc2_table, ec2_table)
